# Initial kernel scaffold; baseline (speedup 1.0000x reference)
import jax
import jax.numpy as jnp
from jax.experimental import pallas as pl


def kernel(W, q_len, k_len):
    raise NotImplementedError("write your pallas kernel here")


# speedup vs baseline: 103.1096x; 103.1096x over previous
import math

import jax
import jax.numpy as jnp
from jax.experimental import pallas as pl
from jax.experimental.pallas import tpu as pltpu

_NUM_HEADS = 16
_NUM_BUCKETS = 32
_MAX_DISTANCE = 128
_QL = 2048
_KL = 2048

_S = 128
_MW = 4352


def _bucket_values(d, w_smem, h):
    nb = _NUM_BUCKETS // 2
    me = nb // 2
    side = jnp.where(d > 0, nb, 0)
    a = jnp.abs(d)
    af = jnp.maximum(a, me).astype(jnp.float32)
    large = me + (
        jnp.log(af / float(me)) / math.log(_MAX_DISTANCE / me) * (nb - me)
    ).astype(jnp.int32)
    large = jnp.minimum(large, nb - 1)
    bucket = side + jnp.where(a < me, a, large)
    acc = jnp.zeros(d.shape, jnp.float32)
    for b in range(_NUM_BUCKETS):
        acc = jnp.where(bucket == b, w_smem[b, h], acc)
    return acc


def _bias_kernel(w_smem, delta_smem, out_ref, m_ref):
    h = pl.program_id(0)
    i = pl.program_id(1)

    @pl.when(i == 0)
    def _build_table():
        j = jax.lax.broadcasted_iota(jnp.int32, (1, _MW), 1)
        d = j - (2047 - delta_smem[0])
        vbase = _bucket_values(d, w_smem, h)
        rows = jnp.broadcast_to(vbase, (_S, _MW))
        m_ref[...] = pltpu.roll(
            rows, _MW - 127, axis=1, stride=1, stride_axis=0
        )

    m0 = pl.multiple_of(_S * (_QL // _S - 1 - i), _S)
    out_ref[0, 0, :, :] = m_ref[:, pl.ds(m0, _KL)]


def kernel(W, q_len, k_len):
    delta = (
        jnp.asarray(k_len, jnp.int32) - _KL
        - (jnp.asarray(q_len, jnp.int32) - _QL)
    ).reshape((1,))
    grid = (_NUM_HEADS, _QL // _S)
    out = pl.pallas_call(
        _bias_kernel,
        grid=grid,
        in_specs=[
            pl.BlockSpec(memory_space=pltpu.SMEM),
            pl.BlockSpec(memory_space=pltpu.SMEM),
        ],
        out_specs=pl.BlockSpec(
            (1, 1, _S, _KL), lambda h, i: (0, h, i, 0)
        ),
        out_shape=jax.ShapeDtypeStruct((1, _NUM_HEADS, _QL, _KL), jnp.float32),
        scratch_shapes=[pltpu.VMEM((_S, _MW), jnp.float32)],
    )(W, delta)
    return out


# speedup vs baseline: 186.5937x; 1.8097x over previous
import math

import jax
import jax.numpy as jnp
from jax.experimental import pallas as pl
from jax.experimental.pallas import tpu as pltpu

_NUM_HEADS = 16
_NUM_BUCKETS = 32
_MAX_DISTANCE = 128
_QL = 2048
_KL = 2048

_S = 256
_NSTRIP = _QL // _S
_MW = 4352


def _bucket_values(d, w_smem, h):
    nb = _NUM_BUCKETS // 2
    me = nb // 2
    side = jnp.where(d > 0, nb, 0)
    a = jnp.abs(d)
    af = jnp.maximum(a, me).astype(jnp.float32)
    large = me + (
        jnp.log(af / float(me)) / math.log(_MAX_DISTANCE / me) * (nb - me)
    ).astype(jnp.int32)
    large = jnp.minimum(large, nb - 1)
    bucket = side + jnp.where(a < me, a, large)
    acc = jnp.zeros(d.shape, jnp.float32)
    for b in range(_NUM_BUCKETS):
        acc = jnp.where(bucket == b, w_smem[b, h], acc)
    return acc


def _bias_kernel(w_smem, delta_smem, out_ref, m_ref, sem_ref):
    h = pl.program_id(0)
    last = pl.num_programs(0) - 1
    buf = h % 2

    def strip_copies(b, hh):
        return [
            pltpu.make_async_copy(
                m_ref.at[b, :, pl.ds(_S * (_NSTRIP - 1 - i), _KL)],
                out_ref.at[0, hh, pl.ds(_S * i, _S), :],
                sem_ref.at[b],
            )
            for i in range(_NSTRIP)
        ]

    @pl.when(h >= 2)
    def _wait_prev():
        for c in strip_copies(buf, h - 2):
            c.wait()

    j = jax.lax.broadcasted_iota(jnp.int32, (1, _MW), 1)
    d = j - (2047 - delta_smem[0])
    vbase = _bucket_values(d, w_smem, h)
    m_ref[buf] = pltpu.roll(
        jnp.broadcast_to(vbase, (_S, _MW)),
        _MW - (_S - 1),
        axis=1,
        stride=1,
        stride_axis=0,
    )

    for c in strip_copies(buf, h):
        c.start()

    @pl.when(h == last)
    def _drain():
        for c in strip_copies(buf, h):
            c.wait()
        for c in strip_copies(1 - buf, h - 1):
            c.wait()


def kernel(W, q_len, k_len):
    delta = (
        jnp.asarray(k_len, jnp.int32) - _KL
        - (jnp.asarray(q_len, jnp.int32) - _QL)
    ).reshape((1,))
    out = pl.pallas_call(
        _bias_kernel,
        grid=(_NUM_HEADS,),
        in_specs=[
            pl.BlockSpec(memory_space=pltpu.SMEM),
            pl.BlockSpec(memory_space=pltpu.SMEM),
        ],
        out_specs=pl.BlockSpec(memory_space=pl.ANY),
        out_shape=jax.ShapeDtypeStruct((1, _NUM_HEADS, _QL, _KL), jnp.float32),
        scratch_shapes=[
            pltpu.VMEM((2, _S, _MW), jnp.float32),
            pltpu.SemaphoreType.DMA((2,)),
        ],
    )(W, delta)
    return out


# speedup vs baseline: 186.8678x; 1.0015x over previous
import math

import jax
import jax.numpy as jnp
from jax.experimental import pallas as pl
from jax.experimental.pallas import tpu as pltpu

_NUM_HEADS = 16
_NUM_BUCKETS = 32
_MAX_DISTANCE = 128
_QL = 2048
_KL = 2048

_S = 512
_NSTRIP = _QL // _S
_MW = 4608


def _bucket_values(d, w_smem, h):
    nb = _NUM_BUCKETS // 2
    me = nb // 2
    side = jnp.where(d > 0, nb, 0)
    a = jnp.abs(d)
    af = jnp.maximum(a, me).astype(jnp.float32)
    large = me + (
        jnp.log(af / float(me)) / math.log(_MAX_DISTANCE / me) * (nb - me)
    ).astype(jnp.int32)
    large = jnp.minimum(large, nb - 1)
    bucket = side + jnp.where(a < me, a, large)
    acc = jnp.zeros(d.shape, jnp.float32)
    for b in range(_NUM_BUCKETS):
        acc = jnp.where(bucket == b, w_smem[b, h], acc)
    return acc


def _bias_kernel(w_smem, delta_smem, out_ref, m_ref, sem_ref):
    h = pl.program_id(0)
    last = pl.num_programs(0) - 1
    buf = h % 2

    def strip_copies(b, hh):
        return [
            pltpu.make_async_copy(
                m_ref.at[b, :, pl.ds(_S * (_NSTRIP - 1 - i), _KL)],
                out_ref.at[0, hh, pl.ds(_S * i, _S), :],
                sem_ref.at[b],
            )
            for i in range(_NSTRIP)
        ]

    @pl.when(h >= 2)
    def _wait_prev():
        for c in strip_copies(buf, h - 2):
            c.wait()

    j = jax.lax.broadcasted_iota(jnp.int32, (1, _MW), 1)
    d = j - (2047 - delta_smem[0])
    vbase = _bucket_values(d, w_smem, h)
    m_ref[buf] = pltpu.roll(
        jnp.broadcast_to(vbase, (_S, _MW)),
        _MW - (_S - 1),
        axis=1,
        stride=1,
        stride_axis=0,
    )

    for c in strip_copies(buf, h):
        c.start()

    @pl.when(h == last)
    def _drain():
        for c in strip_copies(buf, h):
            c.wait()
        for c in strip_copies(1 - buf, h - 1):
            c.wait()


def kernel(W, q_len, k_len):
    delta = (
        jnp.asarray(k_len, jnp.int32) - _KL
        - (jnp.asarray(q_len, jnp.int32) - _QL)
    ).reshape((1,))
    out = pl.pallas_call(
        _bias_kernel,
        grid=(_NUM_HEADS,),
        in_specs=[
            pl.BlockSpec(memory_space=pltpu.SMEM),
            pl.BlockSpec(memory_space=pltpu.SMEM),
        ],
        out_specs=pl.BlockSpec(memory_space=pl.ANY),
        out_shape=jax.ShapeDtypeStruct((1, _NUM_HEADS, _QL, _KL), jnp.float32),
        scratch_shapes=[
            pltpu.VMEM((2, _S, _MW), jnp.float32),
            pltpu.SemaphoreType.DMA((2,)),
        ],
    )(W, delta)
    return out
